# baseline (speedup 1.0000x reference)
import functools

import jax
import jax.numpy as jnp
from jax import lax
from jax.experimental import pallas as pl
from jax.experimental.pallas import tpu as pltpu
from jax.experimental.pallas import tpu_sc as plsc

B, N, E = 8, 1024, 32768
NODE_DIM, EDGE_DIM, HIDDEN, HEADS = 128, 16, 128, 4
HEAD_DIM = HIDDEN // HEADS

NSUB = 16
NCORE = 2
EPT = E // NSUB
CHUNK = 128
NCHUNK = EPT // CHUNK
ROWS = N // NSUB
BPC = B // NCORE
DTOT = HEADS * N
DSL = DTOT // NSUB


def _prep_nodes_body(nf_ref, wnt_ref, asd_ref, h_ref, sd_ref):
    h = jnp.dot(nf_ref[0], wnt_ref[...], preferred_element_type=jnp.float32)
    h_ref[...] = h
    sd_ref[0] = lax.dot_general(asd_ref[...], h, (((0,), (1,)), ((), ())),
                                preferred_element_type=jnp.float32)


def _prep_edges_body(eat_ref, wet_ref, ae_ref, e_ref, el_ref):
    e = lax.dot_general(eat_ref[0], wet_ref[...], (((0,), (0,)), ((), ())),
                        preferred_element_type=jnp.float32)
    e_ref[0] = e
    el_ref[0] = lax.dot_general(ae_ref[...], e, (((0,), (1,)), ((), ())),
                                preferred_element_type=jnp.float32)


def _finish_body(agg_ref, den_ref, nf_ref, g_ref, b_ref, o_ref):
    den = den_ref[0, :HEADS].T
    inv = 1.0 / jnp.where(den > 0, den, 1.0)
    invr = jnp.reshape(
        jnp.broadcast_to(inv[:, :, None], (N, HEADS, HEAD_DIM)), (N, HIDDEN))
    res = agg_ref[0] * invr + nf_ref[0]
    mean = jnp.mean(res, axis=1, keepdims=True)
    xc = res - mean
    var = jnp.mean(xc * xc, axis=1, keepdims=True)
    y = xc * lax.rsqrt(var + 1e-5) * g_ref[...] + b_ref[...]
    o_ref[0] = jnp.where(y > 0, y, jnp.exp(y) - 1.0)


_sc_mesh = plsc.VectorSubcoreMesh(core_axis_name="c", subcore_axis_name="s")


@functools.partial(
    pl.kernel,
    out_type=(
        jax.ShapeDtypeStruct((B, N, HIDDEN), jnp.float32),
        jax.ShapeDtypeStruct((B, 8, N), jnp.float32),
    ),
    mesh=_sc_mesh,
    compiler_params=pltpu.CompilerParams(needs_layout_passes=False),
    scratch_types=[
        pltpu.VMEM((NCHUNK, CHUNK), jnp.int32),
        pltpu.VMEM((NCHUNK, CHUNK), jnp.int32),
        pltpu.VMEM((CHUNK,), jnp.int32),
        pltpu.VMEM((CHUNK,), jnp.int32),
        pltpu.VMEM((CHUNK,), jnp.int32),
        pltpu.VMEM((CHUNK,), jnp.int32),
        pltpu.VMEM((N * 8,), jnp.float32),
        pltpu.VMEM((EPT * HEADS,), jnp.float32),
        pltpu.VMEM((EPT * HEADS + 16,), jnp.float32),
        pltpu.VMEM((DTOT,), jnp.float32),
        pltpu.VMEM((NSUB * DSL,), jnp.float32),
        pltpu.VMEM((DSL,), jnp.float32),
        pltpu.VMEM((CHUNK, HIDDEN), jnp.float32),
        pltpu.VMEM((CHUNK, HIDDEN), jnp.float32),
        pltpu.VMEM((CHUNK, HIDDEN), jnp.float32),
        pltpu.VMEM((CHUNK, HIDDEN), jnp.float32),
        pltpu.VMEM((ROWS, HIDDEN), jnp.float32),
        pltpu.VMEM_SHARED((N, HIDDEN), jnp.float32),
        pltpu.VMEM_SHARED((NSUB * DTOT,), jnp.float32),
        pltpu.SemaphoreType.DMA,
        pltpu.SemaphoreType.DMA,
        pltpu.SemaphoreType.DMA,
        pltpu.SemaphoreType.DMA,
        pltpu.SemaphoreType.DMA,
        pltpu.SemaphoreType.DMA,
    ],
)
def _sc_gat(h_hbm, sd_hbm, e_hbm, el_hbm, src_hbm, dst_hbm,
            agg_hbm, den_hbm,
            src_c_v, dst_c_v, gidx0, gidx1, didx0, didx1, sd_v, el_v, w_v,
            den_v, red_v, dsum_v, e_v0, e_v1, hg_v0, hg_v1, z_v,
            agg_sh, den_all_sh,
            sem_e0, sem_e1, sem_g0, sem_g1, sem_s0, sem_s1):
    gidx = (gidx0, gidx1)
    didx = (didx0, didx1)
    e_v = (e_v0, e_v1)
    hg_v = (hg_v0, hg_v1)
    sem_e = (sem_e0, sem_e1)
    sem_g = (sem_g0, sem_g1)
    sem_s = (sem_s0, sem_s1)
    cid = lax.axis_index("c")
    sid = lax.axis_index("s")

    pltpu.sync_copy(src_hbm.at[sid], src_c_v)
    pltpu.sync_copy(dst_hbm.at[sid], dst_c_v)

    def _zz(i, _):
        for j in range(HIDDEN // 16):
            z_v[i, pl.ds(j * 16, 16)] = jnp.zeros((16,), jnp.float32)
        return 0
    lax.fori_loop(0, ROWS, _zz, 0)

    def batch_body(bl, _):
        b = cid * BPC + bl

        pltpu.sync_copy(sd_hbm.at[b], sd_v)
        for h in range(HEADS):
            pltpu.sync_copy(el_hbm.at[b, h, pl.ds(sid * EPT, EPT)],
                            el_v.at[pl.ds(h * EPT, EPT)])

        def _zd(i, _):
            den_v[pl.ds(i * 16, 16)] = jnp.zeros((16,), jnp.float32)
            return 0
        lax.fori_loop(0, DTOT // 16, _zd, 0)
        pltpu.sync_copy(z_v, agg_sh.at[pl.ds(sid * ROWS, ROWS)])
        plsc.subcore_barrier()

        def phase_a(g, _):
            c = g // (CHUNK // 16)
            o = (g % (CHUNK // 16)) * 16
            src16 = src_c_v[c, pl.ds(o, 16)]
            dst16 = dst_c_v[c, pl.ds(o, 16)]
            k16 = g * 16 + lax.iota(jnp.int32, 16)
            for h in range(HEADS):
                sv = plsc.load_gather(sd_v, [src16 + h * N])
                dv = plsc.load_gather(sd_v, [dst16 + (HEADS + h) * N])
                ev = el_v[pl.ds(h * EPT + g * 16, 16)]
                l = sv + dv + ev
                l = jnp.where(l >= 0, l, l * jnp.float32(0.2))
                w = jnp.exp(l)
                plsc.store_scatter(w_v, [k16 * HEADS + h], w)
                plsc.addupdate_scatter(den_v, [dst16 + h * N], w)
            return 0
        lax.fori_loop(0, EPT // 16, phase_a, 0)

        base = b * N

        def _issue(s, p):
            for j in range(CHUNK // 16):
                gidx[p][pl.ds(j * 16, 16)] = src_c_v[s, pl.ds(j * 16, 16)] + base
                didx[p][pl.ds(j * 16, 16)] = dst_c_v[s, pl.ds(j * 16, 16)]
            de = pltpu.async_copy(
                e_hbm.at[b, pl.ds(sid * EPT + s * CHUNK, CHUNK)], e_v[p], sem_e[p])
            dg = pltpu.async_copy(h_hbm.at[gidx[p]], hg_v[p], sem_g[p])
            return de, dg

        in_flight = {0: _issue(0, 0), 1: None}
        for s in range(NCHUNK):
            p = s % 2
            q = 1 - p
            if s + 1 < NCHUNK:
                in_flight[q] = _issue(s + 1, q)
            de, dg = in_flight[p]
            de.wait()
            dg.wait()

            def edge_body(k, _, _s=s, _p=p):
                wrow = w_v[pl.ds((_s * CHUNK + k) * HEADS, 16)]
                for h in range(HEADS):
                    wb = jnp.full((16,), wrow[h])
                    for j2 in range(HEAD_DIM // 16):
                        col = h * HEAD_DIM + j2 * 16
                        m = (hg_v[_p][k, pl.ds(col, 16)]
                             + e_v[_p][k, pl.ds(col, 16)]) * wb
                        hg_v[_p][k, pl.ds(col, 16)] = m
                return 0
            lax.fori_loop(0, CHUNK, edge_body, 0)

            pltpu.sync_copy(hg_v[p], agg_sh.at[didx[p]], add=True)

        pltpu.sync_copy(den_v, den_all_sh.at[pl.ds(sid * DTOT, DTOT)])
        plsc.subcore_barrier()

        pltpu.sync_copy(agg_sh.at[pl.ds(sid * ROWS, ROWS)],
                        agg_hbm.at[b, pl.ds(sid * ROWS, ROWS)])
        for t in range(NSUB):
            pltpu.sync_copy(den_all_sh.at[pl.ds(t * DTOT + sid * DSL, DSL)],
                            red_v.at[pl.ds(t * DSL, DSL)])
        for j in range(DSL // 16):
            acc = red_v[pl.ds(j * 16, 16)]
            for t in range(1, NSUB):
                acc = acc + red_v[pl.ds(t * DSL + j * 16, 16)]
            dsum_v[pl.ds(j * 16, 16)] = acc
        pltpu.sync_copy(dsum_v,
                        den_hbm.at[b, sid // (N // DSL),
                                   pl.ds((sid % (N // DSL)) * DSL, DSL)])
        plsc.subcore_barrier()
        return 0

    lax.fori_loop(0, BPC, batch_body, 0)


def kernel(node_feat, edge_index, edge_attr, W_node, W_edge,
           att_src, att_dst, att_edge, ln_gamma, ln_beta):
    f32 = jnp.float32
    eye = jnp.eye(HEADS, dtype=f32)
    a_src = (eye[:, None, :] * att_src[:, :, None]).reshape(HIDDEN, HEADS)
    a_dst = (eye[:, None, :] * att_dst[:, :, None]).reshape(HIDDEN, HEADS)
    a_edge = (eye[:, None, :] * att_edge[:, :, None]).reshape(HIDDEN, HEADS)
    a_sd = jnp.concatenate([a_src, a_dst], axis=1)

    h, sd = pl.pallas_call(
        _prep_nodes_body,
        grid=(B,),
        in_specs=[
            pl.BlockSpec((1, N, NODE_DIM), lambda b: (b, 0, 0)),
            pl.BlockSpec((NODE_DIM, HIDDEN), lambda b: (0, 0)),
            pl.BlockSpec((HIDDEN, 2 * HEADS), lambda b: (0, 0)),
        ],
        out_specs=[
            pl.BlockSpec((N, HIDDEN), lambda b: (b, 0)),
            pl.BlockSpec((1, 2 * HEADS, N), lambda b: (b, 0, 0)),
        ],
        out_shape=[
            jax.ShapeDtypeStruct((B * N, HIDDEN), f32),
            jax.ShapeDtypeStruct((B, 2 * HEADS, N), f32),
        ],
    )(node_feat, W_node.T, a_sd)
    sd = sd.reshape(B, 2 * HEADS * N)

    ECH = 4096
    a_edge8 = jnp.concatenate([a_edge, jnp.zeros((HIDDEN, HEADS), f32)], axis=1)
    edge_attr_t = jnp.transpose(edge_attr, (0, 2, 1))
    e, el = pl.pallas_call(
        _prep_edges_body,
        grid=(B, E // ECH),
        in_specs=[
            pl.BlockSpec((1, EDGE_DIM, ECH), lambda b, c: (b, 0, c)),
            pl.BlockSpec((EDGE_DIM, HIDDEN), lambda b, c: (0, 0)),
            pl.BlockSpec((HIDDEN, 2 * HEADS), lambda b, c: (0, 0)),
        ],
        out_specs=[
            pl.BlockSpec((1, ECH, HIDDEN), lambda b, c: (b, c, 0)),
            pl.BlockSpec((1, 2 * HEADS, ECH), lambda b, c: (b, 0, c)),
        ],
        out_shape=[
            jax.ShapeDtypeStruct((B, E, HIDDEN), f32),
            jax.ShapeDtypeStruct((B, 2 * HEADS, E), f32),
        ],
    )(edge_attr_t, W_edge.T, a_edge8)

    src_r = edge_index[0].reshape(NSUB, NCHUNK, CHUNK)
    dst_r = edge_index[1].reshape(NSUB, NCHUNK, CHUNK)

    agg, den = _sc_gat(h, sd, e, el, src_r, dst_r)

    out = pl.pallas_call(
        _finish_body,
        grid=(B,),
        in_specs=[
            pl.BlockSpec((1, N, HIDDEN), lambda b: (b, 0, 0)),
            pl.BlockSpec((1, 8, N), lambda b: (b, 0, 0)),
            pl.BlockSpec((1, N, HIDDEN), lambda b: (b, 0, 0)),
            pl.BlockSpec((1, HIDDEN), lambda b: (0, 0)),
            pl.BlockSpec((1, HIDDEN), lambda b: (0, 0)),
        ],
        out_specs=pl.BlockSpec((1, N, HIDDEN), lambda b: (b, 0, 0)),
        out_shape=jax.ShapeDtypeStruct((B, N, HIDDEN), f32),
    )(agg, den, node_feat, ln_gamma.reshape(1, HIDDEN), ln_beta.reshape(1, HIDDEN))
    return out


# speedup vs baseline: 1.2228x; 1.2228x over previous
import functools

import jax
import jax.numpy as jnp
from jax import lax
from jax.experimental import pallas as pl
from jax.experimental.pallas import tpu as pltpu
from jax.experimental.pallas import tpu_sc as plsc

B, N, E = 8, 1024, 32768
NODE_DIM, EDGE_DIM, HIDDEN, HEADS = 128, 16, 128, 4
HEAD_DIM = HIDDEN // HEADS

NSUB = 16
NCORE = 2
EPT = E // NSUB
CHUNK = 128
NCHUNK = EPT // CHUNK
ROWS = N // NSUB
BPC = B // NCORE
DTOT = HEADS * N
DSL = DTOT // NSUB


def _prep_nodes_body(nf_ref, wnt_ref, asd_ref, h_ref, sd_ref):
    h = jnp.dot(nf_ref[0], wnt_ref[...], preferred_element_type=jnp.float32)
    h_ref[...] = h
    sd_ref[0] = lax.dot_general(asd_ref[...], h, (((0,), (1,)), ((), ())),
                                preferred_element_type=jnp.float32)


def _prep_edges_body(eat_ref, wet_ref, ae_ref, e_ref, el_ref):
    e = lax.dot_general(eat_ref[0], wet_ref[...], (((0,), (0,)), ((), ())),
                        preferred_element_type=jnp.float32)
    e_ref[0] = e
    el_ref[0] = lax.dot_general(ae_ref[...], e, (((0,), (1,)), ((), ())),
                                preferred_element_type=jnp.float32)


def _finish_body(agg_ref, den_ref, nf_ref, g_ref, b_ref, o_ref):
    den = den_ref[0, :HEADS].T
    inv = 1.0 / jnp.where(den > 0, den, 1.0)
    invr = jnp.reshape(
        jnp.broadcast_to(inv[:, :, None], (N, HEADS, HEAD_DIM)), (N, HIDDEN))
    res = agg_ref[0] * invr + nf_ref[0]
    mean = jnp.mean(res, axis=1, keepdims=True)
    xc = res - mean
    var = jnp.mean(xc * xc, axis=1, keepdims=True)
    y = xc * lax.rsqrt(var + 1e-5) * g_ref[...] + b_ref[...]
    o_ref[0] = jnp.where(y > 0, y, jnp.exp(y) - 1.0)


_sc_mesh = plsc.VectorSubcoreMesh(core_axis_name="c", subcore_axis_name="s")


@functools.partial(
    pl.kernel,
    out_type=(
        jax.ShapeDtypeStruct((B, N, HIDDEN), jnp.float32),
        jax.ShapeDtypeStruct((B, 8, N), jnp.float32),
    ),
    mesh=_sc_mesh,
    compiler_params=pltpu.CompilerParams(needs_layout_passes=False),
    scratch_types=[
        pltpu.VMEM((NCHUNK, CHUNK), jnp.int32),
        pltpu.VMEM((NCHUNK, CHUNK), jnp.int32),
        pltpu.VMEM((CHUNK,), jnp.int32),
        pltpu.VMEM((CHUNK,), jnp.int32),
        pltpu.VMEM((CHUNK,), jnp.int32),
        pltpu.VMEM((CHUNK,), jnp.int32),
        pltpu.VMEM((N * 8,), jnp.float32),
        pltpu.VMEM((EPT * HEADS,), jnp.float32),
        pltpu.VMEM((EPT * HEADS + 16,), jnp.float32),
        pltpu.VMEM((DTOT,), jnp.float32),
        pltpu.VMEM((NSUB * DSL,), jnp.float32),
        pltpu.VMEM((DSL,), jnp.float32),
        pltpu.VMEM((CHUNK, HIDDEN), jnp.float32),
        pltpu.VMEM((CHUNK, HIDDEN), jnp.float32),
        pltpu.VMEM((CHUNK, HIDDEN), jnp.float32),
        pltpu.VMEM((CHUNK, HIDDEN), jnp.float32),
        pltpu.VMEM((ROWS, HIDDEN), jnp.float32),
        pltpu.VMEM_SHARED((N, HIDDEN), jnp.float32),
        pltpu.VMEM_SHARED((NSUB * DTOT,), jnp.float32),
        pltpu.SemaphoreType.DMA,
        pltpu.SemaphoreType.DMA,
        pltpu.SemaphoreType.DMA,
        pltpu.SemaphoreType.DMA,
        pltpu.SemaphoreType.DMA,
        pltpu.SemaphoreType.DMA,
    ],
)
def _sc_gat(h_hbm, sd_hbm, e_hbm, el_hbm, src_hbm, dst_hbm,
            agg_hbm, den_hbm,
            src_c_v, dst_c_v, gidx0, gidx1, didx0, didx1, sd_v, el_v, w_v,
            den_v, red_v, dsum_v, e_v0, e_v1, hg_v0, hg_v1, z_v,
            agg_sh, den_all_sh,
            sem_e0, sem_e1, sem_g0, sem_g1, sem_s0, sem_s1):
    gidx = (gidx0, gidx1)
    didx = (didx0, didx1)
    e_v = (e_v0, e_v1)
    hg_v = (hg_v0, hg_v1)
    sem_e = (sem_e0, sem_e1)
    sem_g = (sem_g0, sem_g1)
    sem_s = (sem_s0, sem_s1)
    cid = lax.axis_index("c")
    sid = lax.axis_index("s")

    pltpu.sync_copy(src_hbm.at[sid], src_c_v)
    pltpu.sync_copy(dst_hbm.at[sid], dst_c_v)

    def _zz(i, _):
        for j in range(HIDDEN // 16):
            z_v[i, pl.ds(j * 16, 16)] = jnp.zeros((16,), jnp.float32)
        return 0
    lax.fori_loop(0, ROWS, _zz, 0)

    def batch_body(bl, _):
        b = cid * BPC + bl

        pltpu.sync_copy(sd_hbm.at[b], sd_v)
        for h in range(HEADS):
            pltpu.sync_copy(el_hbm.at[b, h, pl.ds(sid * EPT, EPT)],
                            el_v.at[pl.ds(h * EPT, EPT)])

        def _zd(i, _):
            den_v[pl.ds(i * 16, 16)] = jnp.zeros((16,), jnp.float32)
            return 0
        lax.fori_loop(0, DTOT // 16, _zd, 0)
        pltpu.sync_copy(z_v, agg_sh.at[pl.ds(sid * ROWS, ROWS)])
        plsc.subcore_barrier()

        def phase_a(g, _):
            c = g // (CHUNK // 16)
            o = (g % (CHUNK // 16)) * 16
            src16 = src_c_v[c, pl.ds(o, 16)]
            dst16 = dst_c_v[c, pl.ds(o, 16)]
            k16 = g * 16 + lax.iota(jnp.int32, 16)
            for h in range(HEADS):
                sv = plsc.load_gather(sd_v, [src16 + h * N])
                dv = plsc.load_gather(sd_v, [dst16 + (HEADS + h) * N])
                ev = el_v[pl.ds(h * EPT + g * 16, 16)]
                l = sv + dv + ev
                l = jnp.where(l >= 0, l, l * jnp.float32(0.2))
                w = jnp.exp(l)
                plsc.store_scatter(w_v, [k16 * HEADS + h], w)
                plsc.addupdate_scatter(den_v, [dst16 + h * N], w)
            return 0
        lax.fori_loop(0, EPT // 16, phase_a, 0)

        base = b * N

        def _issue(s, p):
            for j in range(CHUNK // 16):
                gidx[p][pl.ds(j * 16, 16)] = src_c_v[s, pl.ds(j * 16, 16)] + base
                didx[p][pl.ds(j * 16, 16)] = dst_c_v[s, pl.ds(j * 16, 16)]
            pltpu.async_copy(
                e_hbm.at[b, pl.ds(sid * EPT + s * CHUNK, CHUNK)], e_v[p], sem_e[p])
            pltpu.async_copy(h_hbm.at[gidx[p]], hg_v[p], sem_g[p])

        def _wait_in(p):
            pltpu.make_async_copy(
                e_hbm.at[b, pl.ds(sid * EPT, CHUNK)], e_v[p], sem_e[p]).wait()
            pltpu.make_async_copy(h_hbm.at[gidx[p]], hg_v[p], sem_g[p]).wait()

        def _compute_scatter(s, p):
            def edge_body(k, _):
                wrow = w_v[pl.ds((s * CHUNK + k) * HEADS, 16)]
                for h in range(HEADS):
                    wb = jnp.full((16,), wrow[h])
                    for j2 in range(HEAD_DIM // 16):
                        col = h * HEAD_DIM + j2 * 16
                        m = (hg_v[p][k, pl.ds(col, 16)]
                             + e_v[p][k, pl.ds(col, 16)]) * wb
                        hg_v[p][k, pl.ds(col, 16)] = m
                return 0
            lax.fori_loop(0, CHUNK, edge_body, 0)
            pltpu.sync_copy(hg_v[p], agg_sh.at[didx[p]], add=True)

        _issue(0, 0)

        def super_body(t, _):
            _issue(2 * t + 1, 1)
            _wait_in(0)
            _compute_scatter(2 * t, 0)

            @pl.when(t < NCHUNK // 2 - 1)
            def _():
                _issue(2 * t + 2, 0)
            _wait_in(1)
            _compute_scatter(2 * t + 1, 1)
            return 0
        lax.fori_loop(0, NCHUNK // 2, super_body, 0)

        pltpu.sync_copy(den_v, den_all_sh.at[pl.ds(sid * DTOT, DTOT)])
        plsc.subcore_barrier()

        pltpu.sync_copy(agg_sh.at[pl.ds(sid * ROWS, ROWS)],
                        agg_hbm.at[b, pl.ds(sid * ROWS, ROWS)])
        for t in range(NSUB):
            pltpu.sync_copy(den_all_sh.at[pl.ds(t * DTOT + sid * DSL, DSL)],
                            red_v.at[pl.ds(t * DSL, DSL)])
        for j in range(DSL // 16):
            acc = red_v[pl.ds(j * 16, 16)]
            for t in range(1, NSUB):
                acc = acc + red_v[pl.ds(t * DSL + j * 16, 16)]
            dsum_v[pl.ds(j * 16, 16)] = acc
        pltpu.sync_copy(dsum_v,
                        den_hbm.at[b, sid // (N // DSL),
                                   pl.ds((sid % (N // DSL)) * DSL, DSL)])
        plsc.subcore_barrier()
        return 0

    lax.fori_loop(0, BPC, batch_body, 0)


def kernel(node_feat, edge_index, edge_attr, W_node, W_edge,
           att_src, att_dst, att_edge, ln_gamma, ln_beta):
    f32 = jnp.float32
    eye = jnp.eye(HEADS, dtype=f32)
    a_src = (eye[:, None, :] * att_src[:, :, None]).reshape(HIDDEN, HEADS)
    a_dst = (eye[:, None, :] * att_dst[:, :, None]).reshape(HIDDEN, HEADS)
    a_edge = (eye[:, None, :] * att_edge[:, :, None]).reshape(HIDDEN, HEADS)
    a_sd = jnp.concatenate([a_src, a_dst], axis=1)

    h, sd = pl.pallas_call(
        _prep_nodes_body,
        grid=(B,),
        in_specs=[
            pl.BlockSpec((1, N, NODE_DIM), lambda b: (b, 0, 0)),
            pl.BlockSpec((NODE_DIM, HIDDEN), lambda b: (0, 0)),
            pl.BlockSpec((HIDDEN, 2 * HEADS), lambda b: (0, 0)),
        ],
        out_specs=[
            pl.BlockSpec((N, HIDDEN), lambda b: (b, 0)),
            pl.BlockSpec((1, 2 * HEADS, N), lambda b: (b, 0, 0)),
        ],
        out_shape=[
            jax.ShapeDtypeStruct((B * N, HIDDEN), f32),
            jax.ShapeDtypeStruct((B, 2 * HEADS, N), f32),
        ],
    )(node_feat, W_node.T, a_sd)
    sd = sd.reshape(B, 2 * HEADS * N)

    ECH = 4096
    a_edge8 = jnp.concatenate([a_edge, jnp.zeros((HIDDEN, HEADS), f32)], axis=1)
    edge_attr_t = jnp.transpose(edge_attr, (0, 2, 1))
    e, el = pl.pallas_call(
        _prep_edges_body,
        grid=(B, E // ECH),
        in_specs=[
            pl.BlockSpec((1, EDGE_DIM, ECH), lambda b, c: (b, 0, c)),
            pl.BlockSpec((EDGE_DIM, HIDDEN), lambda b, c: (0, 0)),
            pl.BlockSpec((HIDDEN, 2 * HEADS), lambda b, c: (0, 0)),
        ],
        out_specs=[
            pl.BlockSpec((1, ECH, HIDDEN), lambda b, c: (b, c, 0)),
            pl.BlockSpec((1, 2 * HEADS, ECH), lambda b, c: (b, 0, c)),
        ],
        out_shape=[
            jax.ShapeDtypeStruct((B, E, HIDDEN), f32),
            jax.ShapeDtypeStruct((B, 2 * HEADS, E), f32),
        ],
    )(edge_attr_t, W_edge.T, a_edge8)

    src_r = edge_index[0].reshape(NSUB, NCHUNK, CHUNK)
    dst_r = edge_index[1].reshape(NSUB, NCHUNK, CHUNK)

    agg, den = _sc_gat(h, sd, e, el, src_r, dst_r)

    out = pl.pallas_call(
        _finish_body,
        grid=(B,),
        in_specs=[
            pl.BlockSpec((1, N, HIDDEN), lambda b: (b, 0, 0)),
            pl.BlockSpec((1, 8, N), lambda b: (b, 0, 0)),
            pl.BlockSpec((1, N, HIDDEN), lambda b: (b, 0, 0)),
            pl.BlockSpec((1, HIDDEN), lambda b: (0, 0)),
            pl.BlockSpec((1, HIDDEN), lambda b: (0, 0)),
        ],
        out_specs=pl.BlockSpec((1, N, HIDDEN), lambda b: (b, 0, 0)),
        out_shape=jax.ShapeDtypeStruct((B, N, HIDDEN), f32),
    )(agg, den, node_feat, ln_gamma.reshape(1, HIDDEN), ln_beta.reshape(1, HIDDEN))
    return out


# speedup vs baseline: 1.4006x; 1.1454x over previous
import functools

import jax
import jax.numpy as jnp
from jax import lax
from jax.experimental import pallas as pl
from jax.experimental.pallas import tpu as pltpu
from jax.experimental.pallas import tpu_sc as plsc

B, N, E = 8, 1024, 32768
NODE_DIM, EDGE_DIM, HIDDEN, HEADS = 128, 16, 128, 4
HEAD_DIM = HIDDEN // HEADS

NSUB = 16
NCORE = 2
EPT = E // NSUB
CHUNK = 128
NCHUNK = EPT // CHUNK
ROWS = N // NSUB
BPC = B // NCORE
DTOT = HEADS * N
DSL = DTOT // NSUB
HW = HIDDEN // 2


def _prep_nodes_body(nf_ref, wnt_ref, asd_ref, h_ref, sd_ref):
    h = jnp.dot(nf_ref[0], wnt_ref[...], preferred_element_type=jnp.float32)
    h_ref[...] = h
    sd_ref[0] = lax.dot_general(asd_ref[...], h, (((0,), (1,)), ((), ())),
                                preferred_element_type=jnp.float32)


def _prep_edges_body(eat_ref, wet_ref, ae_ref, e_ref, el_ref):
    e = lax.dot_general(eat_ref[0], wet_ref[...], (((0,), (0,)), ((), ())),
                        preferred_element_type=jnp.float32)
    bits = lax.bitcast_convert_type(e, jnp.uint32)
    r = (bits + jnp.uint32(0x7FFF) + ((bits >> 16) & jnp.uint32(1))) >> 16
    word = r[:, :HW] | (r[:, HW:] << 16)
    e_ref[0] = lax.bitcast_convert_type(word, jnp.int32)
    el_ref[0] = lax.dot_general(ae_ref[...], e, (((0,), (1,)), ((), ())),
                                preferred_element_type=jnp.float32)


def _finish_body(agg_ref, den_ref, nf_ref, g_ref, b_ref, o_ref):
    den = den_ref[0, :HEADS].T
    inv = 1.0 / jnp.where(den > 0, den, 1.0)
    invr = jnp.reshape(
        jnp.broadcast_to(inv[:, :, None], (N, HEADS, HEAD_DIM)), (N, HIDDEN))
    res = agg_ref[0] * invr + nf_ref[0]
    mean = jnp.mean(res, axis=1, keepdims=True)
    xc = res - mean
    var = jnp.mean(xc * xc, axis=1, keepdims=True)
    y = xc * lax.rsqrt(var + 1e-5) * g_ref[...] + b_ref[...]
    o_ref[0] = jnp.where(y > 0, y, jnp.exp(y) - 1.0)


_sc_mesh = plsc.VectorSubcoreMesh(core_axis_name="c", subcore_axis_name="s")


@functools.partial(
    pl.kernel,
    out_type=(
        jax.ShapeDtypeStruct((B, N, HIDDEN), jnp.float32),
        jax.ShapeDtypeStruct((B, 8, N), jnp.float32),
    ),
    mesh=_sc_mesh,
    compiler_params=pltpu.CompilerParams(needs_layout_passes=False),
    scratch_types=[
        pltpu.VMEM((NCHUNK, CHUNK), jnp.int32),
        pltpu.VMEM((NCHUNK, CHUNK), jnp.int32),
        pltpu.VMEM((CHUNK,), jnp.int32),
        pltpu.VMEM((CHUNK,), jnp.int32),
        pltpu.VMEM((CHUNK,), jnp.int32),
        pltpu.VMEM((CHUNK,), jnp.int32),
        pltpu.VMEM((N * 8,), jnp.float32),
        pltpu.VMEM((EPT * HEADS,), jnp.float32),
        pltpu.VMEM((EPT * HEADS + 16,), jnp.float32),
        pltpu.VMEM((DTOT,), jnp.float32),
        pltpu.VMEM((NSUB * DSL,), jnp.float32),
        pltpu.VMEM((DSL,), jnp.float32),
        pltpu.VMEM((CHUNK, HW), jnp.int32),
        pltpu.VMEM((CHUNK, HW), jnp.int32),
        pltpu.VMEM((CHUNK, HIDDEN), jnp.float32),
        pltpu.VMEM((CHUNK, HIDDEN), jnp.float32),
        pltpu.VMEM((ROWS, HIDDEN), jnp.float32),
        pltpu.VMEM_SHARED((N, HIDDEN), jnp.float32),
        pltpu.VMEM_SHARED((NSUB * DTOT,), jnp.float32),
        pltpu.SemaphoreType.DMA,
        pltpu.SemaphoreType.DMA,
        pltpu.SemaphoreType.DMA,
        pltpu.SemaphoreType.DMA,
        pltpu.SemaphoreType.DMA,
        pltpu.SemaphoreType.DMA,
    ],
)
def _sc_gat(h_hbm, sd_hbm, e_hbm, el_hbm, src_hbm, dst_hbm,
            agg_hbm, den_hbm,
            src_c_v, dst_c_v, gidx0, gidx1, didx0, didx1, sd_v, el_v, w_v,
            den_v, red_v, dsum_v, e_v0, e_v1, hg_v0, hg_v1, z_v,
            agg_sh, den_all_sh,
            sem_e0, sem_e1, sem_g0, sem_g1, sem_s0, sem_s1):
    gidx = (gidx0, gidx1)
    didx = (didx0, didx1)
    e_v = (e_v0, e_v1)
    hg_v = (hg_v0, hg_v1)
    sem_e = (sem_e0, sem_e1)
    sem_g = (sem_g0, sem_g1)
    cid = lax.axis_index("c")
    sid = lax.axis_index("s")

    pltpu.sync_copy(src_hbm.at[sid], src_c_v)
    pltpu.sync_copy(dst_hbm.at[sid], dst_c_v)

    def _zz(i, _):
        for j in range(HIDDEN // 16):
            z_v[i, pl.ds(j * 16, 16)] = jnp.zeros((16,), jnp.float32)
        return 0
    lax.fori_loop(0, ROWS, _zz, 0)

    def batch_body(bl, _):
        b = cid * BPC + bl

        pltpu.sync_copy(sd_hbm.at[b], sd_v)
        for h in range(HEADS):
            pltpu.sync_copy(el_hbm.at[b, h, pl.ds(sid * EPT, EPT)],
                            el_v.at[pl.ds(h * EPT, EPT)])

        def _zd(i, _):
            den_v[pl.ds(i * 16, 16)] = jnp.zeros((16,), jnp.float32)
            return 0
        lax.fori_loop(0, DTOT // 16, _zd, 0)
        pltpu.sync_copy(z_v, agg_sh.at[pl.ds(sid * ROWS, ROWS)])
        plsc.subcore_barrier()

        def phase_a(g, _):
            c = g // (CHUNK // 16)
            o = (g % (CHUNK // 16)) * 16
            src16 = src_c_v[c, pl.ds(o, 16)]
            dst16 = dst_c_v[c, pl.ds(o, 16)]
            k16 = g * 16 + lax.iota(jnp.int32, 16)
            for h in range(HEADS):
                sv = plsc.load_gather(sd_v, [src16 + h * N])
                dv = plsc.load_gather(sd_v, [dst16 + (HEADS + h) * N])
                ev = el_v[pl.ds(h * EPT + g * 16, 16)]
                l = sv + dv + ev
                l = jnp.where(l >= 0, l, l * jnp.float32(0.2))
                w = jnp.exp(l)
                plsc.store_scatter(w_v, [k16 * HEADS + h], w)
                plsc.addupdate_scatter(den_v, [dst16 + h * N], w)
            return 0
        lax.fori_loop(0, EPT // 16, phase_a, 0)

        base = b * N

        def _issue(s, p):
            for j in range(CHUNK // 16):
                gidx[p][pl.ds(j * 16, 16)] = src_c_v[s, pl.ds(j * 16, 16)] + base
                didx[p][pl.ds(j * 16, 16)] = dst_c_v[s, pl.ds(j * 16, 16)]
            pltpu.async_copy(
                e_hbm.at[b, pl.ds(sid * EPT + s * CHUNK, CHUNK)], e_v[p], sem_e[p])
            pltpu.async_copy(h_hbm.at[gidx[p]], hg_v[p], sem_g[p])

        def _wait_in(p):
            pltpu.make_async_copy(
                e_hbm.at[b, pl.ds(sid * EPT, CHUNK)], e_v[p], sem_e[p]).wait()
            pltpu.make_async_copy(h_hbm.at[gidx[p]], hg_v[p], sem_g[p]).wait()

        def _compute_scatter(s, p):
            def edge_body(k, _):
                wrow = w_v[pl.ds((s * CHUNK + k) * HEADS, 16)]
                ws = [jnp.full((16,), wrow[h]) for h in range(HEADS)]
                for g in range(HW // 16):
                    eb = plsc.bitcast(e_v[p][k, pl.ds(g * 16, 16)],
                                      jnp.bfloat16)
                    lo, hi = plsc.unpack(eb, format=plsc.PackFormat.INTERLEAVED)
                    cl = g * 16
                    ch = HW + g * 16
                    ml = (hg_v[p][k, pl.ds(cl, 16)] + lo) * ws[cl // HEAD_DIM]
                    mh = (hg_v[p][k, pl.ds(ch, 16)] + hi) * ws[ch // HEAD_DIM]
                    hg_v[p][k, pl.ds(cl, 16)] = ml
                    hg_v[p][k, pl.ds(ch, 16)] = mh
                return 0
            lax.fori_loop(0, CHUNK, edge_body, 0)
            pltpu.sync_copy(hg_v[p], agg_sh.at[didx[p]], add=True)

        _issue(0, 0)

        def super_body(t, _):
            _issue(2 * t + 1, 1)
            _wait_in(0)
            _compute_scatter(2 * t, 0)

            @pl.when(t < NCHUNK // 2 - 1)
            def _():
                _issue(2 * t + 2, 0)
            _wait_in(1)
            _compute_scatter(2 * t + 1, 1)
            return 0
        lax.fori_loop(0, NCHUNK // 2, super_body, 0)

        pltpu.sync_copy(den_v, den_all_sh.at[pl.ds(sid * DTOT, DTOT)])
        plsc.subcore_barrier()

        pltpu.sync_copy(agg_sh.at[pl.ds(sid * ROWS, ROWS)],
                        agg_hbm.at[b, pl.ds(sid * ROWS, ROWS)])
        for t in range(NSUB):
            pltpu.sync_copy(den_all_sh.at[pl.ds(t * DTOT + sid * DSL, DSL)],
                            red_v.at[pl.ds(t * DSL, DSL)])
        for j in range(DSL // 16):
            acc = red_v[pl.ds(j * 16, 16)]
            for t in range(1, NSUB):
                acc = acc + red_v[pl.ds(t * DSL + j * 16, 16)]
            dsum_v[pl.ds(j * 16, 16)] = acc
        pltpu.sync_copy(dsum_v,
                        den_hbm.at[b, sid // (N // DSL),
                                   pl.ds((sid % (N // DSL)) * DSL, DSL)])
        plsc.subcore_barrier()
        return 0

    lax.fori_loop(0, BPC, batch_body, 0)


def kernel(node_feat, edge_index, edge_attr, W_node, W_edge,
           att_src, att_dst, att_edge, ln_gamma, ln_beta):
    f32 = jnp.float32
    eye = jnp.eye(HEADS, dtype=f32)
    a_src = (eye[:, None, :] * att_src[:, :, None]).reshape(HIDDEN, HEADS)
    a_dst = (eye[:, None, :] * att_dst[:, :, None]).reshape(HIDDEN, HEADS)
    a_edge = (eye[:, None, :] * att_edge[:, :, None]).reshape(HIDDEN, HEADS)
    a_sd = jnp.concatenate([a_src, a_dst], axis=1)
    a_edge8 = jnp.concatenate([a_edge, jnp.zeros((HIDDEN, HEADS), f32)], axis=1)

    h, sd = pl.pallas_call(
        _prep_nodes_body,
        grid=(B,),
        in_specs=[
            pl.BlockSpec((1, N, NODE_DIM), lambda b: (b, 0, 0)),
            pl.BlockSpec((NODE_DIM, HIDDEN), lambda b: (0, 0)),
            pl.BlockSpec((HIDDEN, 2 * HEADS), lambda b: (0, 0)),
        ],
        out_specs=[
            pl.BlockSpec((N, HIDDEN), lambda b: (b, 0)),
            pl.BlockSpec((1, 2 * HEADS, N), lambda b: (b, 0, 0)),
        ],
        out_shape=[
            jax.ShapeDtypeStruct((B * N, HIDDEN), f32),
            jax.ShapeDtypeStruct((B, 2 * HEADS, N), f32),
        ],
    )(node_feat, W_node.T, a_sd)
    sd = sd.reshape(B, 2 * HEADS * N)

    ECH = 4096
    edge_attr_t = jnp.transpose(edge_attr, (0, 2, 1))
    e, el = pl.pallas_call(
        _prep_edges_body,
        grid=(B, E // ECH),
        in_specs=[
            pl.BlockSpec((1, EDGE_DIM, ECH), lambda b, c: (b, 0, c)),
            pl.BlockSpec((EDGE_DIM, HIDDEN), lambda b, c: (0, 0)),
            pl.BlockSpec((HIDDEN, 2 * HEADS), lambda b, c: (0, 0)),
        ],
        out_specs=[
            pl.BlockSpec((1, ECH, HW), lambda b, c: (b, c, 0)),
            pl.BlockSpec((1, 2 * HEADS, ECH), lambda b, c: (b, 0, c)),
        ],
        out_shape=[
            jax.ShapeDtypeStruct((B, E, HW), jnp.int32),
            jax.ShapeDtypeStruct((B, 2 * HEADS, E), f32),
        ],
    )(edge_attr_t, W_edge.T, a_edge8)

    src_r = edge_index[0].reshape(NSUB, NCHUNK, CHUNK)
    dst_r = edge_index[1].reshape(NSUB, NCHUNK, CHUNK)

    agg, den = _sc_gat(h, sd, e, el, src_r, dst_r)

    out = pl.pallas_call(
        _finish_body,
        grid=(B,),
        in_specs=[
            pl.BlockSpec((1, N, HIDDEN), lambda b: (b, 0, 0)),
            pl.BlockSpec((1, 8, N), lambda b: (b, 0, 0)),
            pl.BlockSpec((1, N, HIDDEN), lambda b: (b, 0, 0)),
            pl.BlockSpec((1, HIDDEN), lambda b: (0, 0)),
            pl.BlockSpec((1, HIDDEN), lambda b: (0, 0)),
        ],
        out_specs=pl.BlockSpec((1, N, HIDDEN), lambda b: (b, 0, 0)),
        out_shape=jax.ShapeDtypeStruct((B, N, HIDDEN), f32),
    )(agg, den, node_feat, ln_gamma.reshape(1, HIDDEN), ln_beta.reshape(1, HIDDEN))
    return out
